# Initial kernel scaffold; baseline (speedup 1.0000x reference)
import functools

import jax
import jax.numpy as jnp
from jax import lax
from jax.experimental import pallas as pl
from jax.experimental.pallas import tpu as pltpu
from jax.experimental.pallas import tpu_sc as plsc

MAP_DIM = 128
N_MAP = 10000
N_AGT = 5000
N_EDGE = 320000
AG_DIM = 80

NC = 2
NS = 16
NW = NC * NS

CHUNK = 128
N_CHUNKS_PAD = 2528
E_PAD = N_CHUNKS_PAD * CHUNK
CHUNKS_PER_W = N_CHUNKS_PAD // NW

ACC_ROWS = 10016
ZROWS = ACC_ROWS // NS
OROWS = N_MAP // NS


def _sc_accumulate(agent_feat, u2, v2, zeros_src):
    mesh = plsc.VectorSubcoreMesh(core_axis_name="c", subcore_axis_name="s")

    @functools.partial(
        pl.kernel,
        out_type=jax.ShapeDtypeStruct((NC * N_MAP, AG_DIM), jnp.float32),
        mesh=mesh,
        scratch_types=[
            pltpu.VMEM((CHUNKS_PER_W, CHUNK), jnp.int32),
            pltpu.VMEM((CHUNKS_PER_W, CHUNK), jnp.int32),
            pltpu.VMEM((CHUNK, AG_DIM), jnp.float32),
            pltpu.VMEM((CHUNK, AG_DIM), jnp.float32),
            pltpu.VMEM_SHARED((ACC_ROWS, AG_DIM), jnp.float32),
            pltpu.SemaphoreType.DMA,
            pltpu.SemaphoreType.DMA,
        ],
    )
    def k(agent_hbm, u_hbm, v_hbm, z_hbm, out_hbm,
          u_t, v_t, rows_a, rows_b, acc, sem_a, sem_b):
        cid = lax.axis_index("c")
        sid = lax.axis_index("s")
        wid = sid * NC + cid

        pltpu.sync_copy(z_hbm, acc.at[pl.ds(sid * ZROWS, ZROWS)])
        pltpu.sync_copy(u_hbm.at[pl.ds(wid * CHUNKS_PER_W, CHUNKS_PER_W)], u_t)
        pltpu.sync_copy(v_hbm.at[pl.ds(wid * CHUNKS_PER_W, CHUNKS_PER_W)], v_t)
        plsc.subcore_barrier()

        pltpu.async_copy(agent_hbm.at[u_t.at[0]], rows_a, sem_a)

        @pl.loop(0, CHUNKS_PER_W - 2, step=2)
        def _(j):
            pltpu.async_copy(agent_hbm.at[u_t.at[j + 1]], rows_b, sem_b)
            pltpu.make_async_copy(agent_hbm.at[u_t.at[j]], rows_a, sem_a).wait()
            pltpu.sync_copy(rows_a, acc.at[v_t.at[j]], add=True)
            pltpu.async_copy(agent_hbm.at[u_t.at[j + 2]], rows_a, sem_a)
            pltpu.make_async_copy(agent_hbm.at[u_t.at[j + 1]], rows_b, sem_b).wait()
            pltpu.sync_copy(rows_b, acc.at[v_t.at[j + 1]], add=True)

        pltpu.async_copy(agent_hbm.at[u_t.at[CHUNKS_PER_W - 1]], rows_b, sem_b)
        pltpu.make_async_copy(
            agent_hbm.at[u_t.at[CHUNKS_PER_W - 2]], rows_a, sem_a).wait()
        pltpu.sync_copy(rows_a, acc.at[v_t.at[CHUNKS_PER_W - 2]], add=True)
        pltpu.make_async_copy(
            agent_hbm.at[u_t.at[CHUNKS_PER_W - 1]], rows_b, sem_b).wait()
        pltpu.sync_copy(rows_b, acc.at[v_t.at[CHUNKS_PER_W - 1]], add=True)

        plsc.subcore_barrier()
        pltpu.sync_copy(
            acc.at[pl.ds(sid * OROWS, OROWS)],
            out_hbm.at[pl.ds(cid * N_MAP + sid * OROWS, OROWS)],
        )

    return k(agent_feat, u2, v2, zeros_src)


BLK = 2000


def _tc_finish_body(f_ref, a0_ref, a1_ref, wm_ref, wa_ref, g_ref, b_ref, o_ref):
    dn = (((1,), (1,)), ((), ()))
    x = lax.dot_general(f_ref[...], wm_ref[...], dn,
                        precision=lax.Precision.HIGHEST,
                        preferred_element_type=jnp.float32)
    a = a0_ref[...] + a1_ref[...]
    x = x + lax.dot_general(a, wa_ref[...], dn,
                            precision=lax.Precision.HIGHEST,
                            preferred_element_type=jnp.float32)
    mean = jnp.mean(x, axis=1, keepdims=True)
    xc = x - mean
    var = jnp.mean(xc * xc, axis=1, keepdims=True)
    xhat = xc / jnp.sqrt(var + 1e-5)
    y = xhat * g_ref[...][None, :] + b_ref[...][None, :]
    o_ref[...] = jnp.maximum(y, 0.0)


def _tc_finish(feats, acc0, acc1, W_map, W_agt, gn_gamma, gn_beta):
    grid = (N_MAP // BLK,)
    return pl.pallas_call(
        _tc_finish_body,
        grid=grid,
        in_specs=[
            pl.BlockSpec((BLK, 8), lambda i: (i, 0)),
            pl.BlockSpec((BLK, AG_DIM), lambda i: (i, 0)),
            pl.BlockSpec((BLK, AG_DIM), lambda i: (i, 0)),
            pl.BlockSpec((MAP_DIM, 8), lambda i: (0, 0)),
            pl.BlockSpec((MAP_DIM, AG_DIM), lambda i: (0, 0)),
            pl.BlockSpec((MAP_DIM,), lambda i: (0,)),
            pl.BlockSpec((MAP_DIM,), lambda i: (0,)),
        ],
        out_specs=pl.BlockSpec((BLK, MAP_DIM), lambda i: (i, 0)),
        out_shape=jax.ShapeDtypeStruct((N_MAP, MAP_DIM), jnp.float32),
    )(feats, acc0, acc1, W_map, W_agt, gn_gamma, gn_beta)


def kernel(feats, agent_feat, a2m_u, a2m_v, W_map, W_agt, gn_gamma, gn_beta):
    u = a2m_u.astype(jnp.int32)
    v = a2m_v.astype(jnp.int32)
    pad = E_PAD - N_EDGE
    u2 = jnp.concatenate([u, jnp.zeros((pad,), jnp.int32)])
    v2 = jnp.concatenate([v, jnp.full((pad,), N_MAP, jnp.int32)])
    u2 = u2.reshape(N_CHUNKS_PAD, CHUNK)
    v2 = v2.reshape(N_CHUNKS_PAD, CHUNK)
    zeros_src = jnp.zeros((ZROWS, AG_DIM), jnp.float32)

    acc = _sc_accumulate(agent_feat, u2, v2, zeros_src)
    return _tc_finish(feats, acc[:N_MAP], acc[N_MAP:], W_map, W_agt,
                      gn_gamma, gn_beta)


# speedup vs baseline: 5.0076x; 5.0076x over previous
import functools

import jax
import jax.numpy as jnp
from jax import lax
from jax.experimental import pallas as pl
from jax.experimental.pallas import tpu as pltpu
from jax.experimental.pallas import tpu_sc as plsc

MAP_DIM = 128
N_MAP = 10000
N_AGT = 5000
N_EDGE = 320000
AG_DIM = 80
AG_PAD = 128

NC = 2
NS = 16

CHUNK = 128
N_CHUNKS = 2560
E_PAD = N_CHUNKS * CHUNK
CPS = N_CHUNKS // NS
BLK_CHUNKS = 32
N_BLKS = CPS // BLK_CHUNKS

HALF = N_MAP // NC
ACC_ROWS = 5120
ZR = ACC_ROWS // NS


def _sc_accumulate(agent_tab, uv):
    mesh = plsc.VectorSubcoreMesh(core_axis_name="c", subcore_axis_name="s")

    @functools.partial(
        pl.kernel,
        out_type=jax.ShapeDtypeStruct((NC * ACC_ROWS, AG_PAD), jnp.float32),
        mesh=mesh,
        scratch_types=[
            pltpu.VMEM((2 * BLK_CHUNKS, CHUNK), jnp.int32),
            pltpu.VMEM((CHUNK,), jnp.int32),
            pltpu.VMEM((CHUNK, AG_PAD), jnp.float32),
            pltpu.VMEM((CHUNK, AG_PAD), jnp.float32),
            pltpu.VMEM_SHARED((N_AGT, AG_PAD), jnp.float32),
            pltpu.VMEM_SHARED((ACC_ROWS, AG_PAD), jnp.float32),
            pltpu.SemaphoreType.DMA,
            pltpu.SemaphoreType.DMA,
        ],
    )
    def k(tab_hbm, uv_hbm, out_hbm, uv_t, vt2, rows_a, rows_b, tab, acc,
          sem_a, sem_b):
        cid = lax.axis_index("c")
        sid = lax.axis_index("s")
        lo = cid * HALF

        @pl.loop(0, CHUNK)
        def _(r):
            @pl.loop(0, AG_PAD, step=16)
            def _(cc):
                rows_a[r, pl.ds(cc, 16)] = jnp.zeros((16,), jnp.float32)

        @pl.loop(0, ZR, step=CHUNK)
        def _(r):
            pltpu.sync_copy(rows_a, acc.at[pl.ds(sid * ZR + r, CHUNK)])
        pltpu.sync_copy(rows_a.at[pl.ds(0, 64)],
                        acc.at[pl.ds(sid * ZR + 256, 64)])

        @pl.when(sid < NS - 1)
        def _():
            pltpu.sync_copy(tab_hbm.at[pl.ds(sid * 312, 312)],
                            tab.at[pl.ds(sid * 312, 312)])

        @pl.when(sid == NS - 1)
        def _():
            pltpu.sync_copy(tab_hbm.at[pl.ds(4680, 320)],
                            tab.at[pl.ds(4680, 320)])

        plsc.subcore_barrier()

        def proc(rows_buf, vrow):
            for t in range(CHUNK // 16):
                x = vrow[pl.ds(t * 16, 16)]
                xs = x - lo
                inr = (xs >= 0) & (xs < HALF)
                vt2[pl.ds(t * 16, 16)] = jnp.where(inr, xs, HALF)
            pltpu.sync_copy(rows_buf, acc.at[vt2], add=True)

        @pl.loop(0, N_BLKS)
        def _(b):
            base = sid * 2 * CPS + b * 2 * BLK_CHUNKS
            pltpu.sync_copy(uv_hbm.at[pl.ds(base, 2 * BLK_CHUNKS)], uv_t)
            pltpu.async_copy(tab.at[uv_t.at[0]], rows_a, sem_a)

            @pl.loop(0, BLK_CHUNKS - 2, step=2)
            def _(j):
                pltpu.async_copy(tab.at[uv_t.at[2 * j + 2]], rows_b, sem_b)
                pltpu.make_async_copy(tab.at[uv_t.at[2 * j]], rows_a,
                                      sem_a).wait()
                proc(rows_a, uv_t.at[2 * j + 1])
                pltpu.async_copy(tab.at[uv_t.at[2 * j + 4]], rows_a, sem_a)
                pltpu.make_async_copy(tab.at[uv_t.at[2 * j + 2]], rows_b,
                                      sem_b).wait()
                proc(rows_b, uv_t.at[2 * j + 3])

            jl = 2 * (BLK_CHUNKS - 1)
            pltpu.async_copy(tab.at[uv_t.at[jl]], rows_b, sem_b)
            pltpu.make_async_copy(tab.at[uv_t.at[jl - 2]], rows_a,
                                  sem_a).wait()
            proc(rows_a, uv_t.at[jl - 1])
            pltpu.make_async_copy(tab.at[uv_t.at[jl]], rows_b, sem_b).wait()
            proc(rows_b, uv_t.at[jl + 1])

        plsc.subcore_barrier()
        pltpu.sync_copy(
            acc.at[pl.ds(sid * ZR, ZR)],
            out_hbm.at[pl.ds(cid * ACC_ROWS + sid * ZR, ZR)],
        )

    return k(agent_tab, uv)


BLK = 2000


def _tc_finish_body(f_ref, a_ref, wm_ref, wa_ref, g_ref, b_ref, o_ref):
    dn = (((1,), (1,)), ((), ()))
    x = lax.dot_general(f_ref[...], wm_ref[...], dn,
                        precision=lax.Precision.HIGHEST,
                        preferred_element_type=jnp.float32)
    x = x + lax.dot_general(a_ref[...], wa_ref[...], dn,
                            precision=lax.Precision.HIGHEST,
                            preferred_element_type=jnp.float32)
    mean = jnp.mean(x, axis=1, keepdims=True)
    xc = x - mean
    var = jnp.mean(xc * xc, axis=1, keepdims=True)
    xhat = xc / jnp.sqrt(var + 1e-5)
    y = xhat * g_ref[...][None, :] + b_ref[...][None, :]
    o_ref[...] = jnp.maximum(y, 0.0)


def _tc_finish(feats, acc, W_map, W_agt_pad, gn_gamma, gn_beta):
    grid = (N_MAP // BLK,)
    return pl.pallas_call(
        _tc_finish_body,
        grid=grid,
        in_specs=[
            pl.BlockSpec((BLK, 8), lambda i: (i, 0)),
            pl.BlockSpec((BLK, AG_PAD), lambda i: (i, 0)),
            pl.BlockSpec((MAP_DIM, 8), lambda i: (0, 0)),
            pl.BlockSpec((MAP_DIM, AG_PAD), lambda i: (0, 0)),
            pl.BlockSpec((MAP_DIM,), lambda i: (0,)),
            pl.BlockSpec((MAP_DIM,), lambda i: (0,)),
        ],
        out_specs=pl.BlockSpec((BLK, MAP_DIM), lambda i: (i, 0)),
        out_shape=jax.ShapeDtypeStruct((N_MAP, MAP_DIM), jnp.float32),
    )(feats, acc, W_map, W_agt_pad, gn_gamma, gn_beta)


def kernel(feats, agent_feat, a2m_u, a2m_v, W_map, W_agt, gn_gamma, gn_beta):
    u = a2m_u.astype(jnp.int32)
    v = a2m_v.astype(jnp.int32)
    pad = E_PAD - N_EDGE
    u2 = jnp.concatenate([u, jnp.zeros((pad,), jnp.int32)])
    v2 = jnp.concatenate([v, jnp.full((pad,), N_MAP, jnp.int32)])
    uv = jnp.stack([u2.reshape(N_CHUNKS, CHUNK),
                    v2.reshape(N_CHUNKS, CHUNK)], axis=1)
    uv = uv.reshape(2 * N_CHUNKS, CHUNK)

    agent_tab = jnp.pad(agent_feat, ((0, 0), (0, AG_PAD - AG_DIM)))
    accs = _sc_accumulate(agent_tab, uv)
    acc = jnp.concatenate([accs[:HALF], accs[ACC_ROWS:ACC_ROWS + HALF]])
    W_agt_pad = jnp.pad(W_agt, ((0, 0), (0, AG_PAD - AG_DIM)))
    return _tc_finish(feats, acc, W_map, W_agt_pad, gn_gamma, gn_beta)


# speedup vs baseline: 5.4594x; 1.0902x over previous
import functools

import jax
import jax.numpy as jnp
from jax import lax
from jax.experimental import pallas as pl
from jax.experimental.pallas import tpu as pltpu
from jax.experimental.pallas import tpu_sc as plsc

MAP_DIM = 128
N_MAP = 10000
N_AGT = 5000
N_EDGE = 320000
AG_DIM = 80
AG_PAD = 128

NC = 2
NS = 16

CHUNK = 128
N_CHUNKS = 2560
E_PAD = N_CHUNKS * CHUNK
CPS = N_CHUNKS // NS
BLK_CHUNKS = 32
N_BLKS = CPS // BLK_CHUNKS

HALF = N_MAP // NC
ACC_ROWS = 5120
ZR = ACC_ROWS // NS


def _sc_accumulate(agent_tab, uv):
    mesh = plsc.VectorSubcoreMesh(core_axis_name="c", subcore_axis_name="s")

    @functools.partial(
        pl.kernel,
        out_type=jax.ShapeDtypeStruct((NC * ACC_ROWS, AG_PAD), jnp.float32),
        mesh=mesh,
        scratch_types=[
            pltpu.VMEM((2 * BLK_CHUNKS, CHUNK), jnp.int32),
            pltpu.VMEM((CHUNK,), jnp.int32),
            pltpu.VMEM((CHUNK, AG_PAD), jnp.float32),
            pltpu.VMEM((CHUNK, AG_PAD), jnp.float32),
            pltpu.VMEM_SHARED((N_AGT, AG_PAD), jnp.float32),
            pltpu.VMEM_SHARED((ACC_ROWS, AG_PAD), jnp.float32),
            pltpu.SemaphoreType.DMA,
            pltpu.SemaphoreType.DMA,
        ],
    )
    def k(tab_hbm, uv_hbm, out_hbm, uv_t, vt2, rows_a, rows_b, tab, acc,
          sem_a, sem_b):
        cid = lax.axis_index("c")
        sid = lax.axis_index("s")
        lo = cid * HALF

        @pl.loop(0, CHUNK)
        def _(r):
            @pl.loop(0, AG_PAD, step=16)
            def _(cc):
                rows_a[r, pl.ds(cc, 16)] = jnp.zeros((16,), jnp.float32)

        @pl.loop(0, ZR, step=CHUNK)
        def _(r):
            pltpu.sync_copy(rows_a, acc.at[pl.ds(sid * ZR + r, CHUNK)])
        pltpu.sync_copy(rows_a.at[pl.ds(0, 64)],
                        acc.at[pl.ds(sid * ZR + 256, 64)])

        @pl.when(sid < NS - 1)
        def _():
            pltpu.sync_copy(tab_hbm.at[pl.ds(sid * 312, 312)],
                            tab.at[pl.ds(sid * 312, 312)])

        @pl.when(sid == NS - 1)
        def _():
            pltpu.sync_copy(tab_hbm.at[pl.ds(4680, 320)],
                            tab.at[pl.ds(4680, 320)])

        plsc.subcore_barrier()

        trash16 = lax.broadcasted_iota(jnp.int32, (16,), 0) + HALF

        def proc(rows_buf, vrow):
            for t in range(CHUNK // 16):
                x = vrow[pl.ds(t * 16, 16)]
                xs = x - lo
                inr = (xs >= 0) & (xs < HALF)
                vt2[pl.ds(t * 16, 16)] = jnp.where(inr, xs,
                                                   trash16 + 16 * (t % 7))
            pltpu.sync_copy(rows_buf, acc.at[vt2], add=True)

        @pl.loop(0, N_BLKS)
        def _(b):
            base = sid * 2 * CPS + b * 2 * BLK_CHUNKS
            pltpu.sync_copy(uv_hbm.at[pl.ds(base, 2 * BLK_CHUNKS)], uv_t)
            pltpu.async_copy(tab.at[uv_t.at[0]], rows_a, sem_a)

            @pl.loop(0, BLK_CHUNKS - 2, step=2)
            def _(j):
                pltpu.async_copy(tab.at[uv_t.at[2 * j + 2]], rows_b, sem_b)
                pltpu.make_async_copy(tab.at[uv_t.at[2 * j]], rows_a,
                                      sem_a).wait()
                proc(rows_a, uv_t.at[2 * j + 1])
                pltpu.async_copy(tab.at[uv_t.at[2 * j + 4]], rows_a, sem_a)
                pltpu.make_async_copy(tab.at[uv_t.at[2 * j + 2]], rows_b,
                                      sem_b).wait()
                proc(rows_b, uv_t.at[2 * j + 3])

            jl = 2 * (BLK_CHUNKS - 1)
            pltpu.async_copy(tab.at[uv_t.at[jl]], rows_b, sem_b)
            pltpu.make_async_copy(tab.at[uv_t.at[jl - 2]], rows_a,
                                  sem_a).wait()
            proc(rows_a, uv_t.at[jl - 1])
            pltpu.make_async_copy(tab.at[uv_t.at[jl]], rows_b, sem_b).wait()
            proc(rows_b, uv_t.at[jl + 1])

        plsc.subcore_barrier()
        pltpu.sync_copy(
            acc.at[pl.ds(sid * ZR, ZR)],
            out_hbm.at[pl.ds(cid * ACC_ROWS + sid * ZR, ZR)],
        )

    return k(agent_tab, uv)


BLK = 2000


def _tc_finish_body(f_ref, a_ref, wm_ref, wa_ref, g_ref, b_ref, o_ref):
    dn = (((1,), (1,)), ((), ()))
    x = lax.dot_general(f_ref[...], wm_ref[...], dn,
                        precision=lax.Precision.HIGHEST,
                        preferred_element_type=jnp.float32)
    x = x + lax.dot_general(a_ref[...], wa_ref[...], dn,
                            precision=lax.Precision.HIGHEST,
                            preferred_element_type=jnp.float32)
    mean = jnp.mean(x, axis=1, keepdims=True)
    xc = x - mean
    var = jnp.mean(xc * xc, axis=1, keepdims=True)
    xhat = xc / jnp.sqrt(var + 1e-5)
    y = xhat * g_ref[...][None, :] + b_ref[...][None, :]
    o_ref[...] = jnp.maximum(y, 0.0)


def _tc_finish(feats, acc, W_map, W_agt_pad, gn_gamma, gn_beta):
    grid = (N_MAP // BLK,)
    return pl.pallas_call(
        _tc_finish_body,
        grid=grid,
        in_specs=[
            pl.BlockSpec((BLK, 8), lambda i: (i, 0)),
            pl.BlockSpec((BLK, AG_PAD), lambda i: (i, 0)),
            pl.BlockSpec((MAP_DIM, 8), lambda i: (0, 0)),
            pl.BlockSpec((MAP_DIM, AG_PAD), lambda i: (0, 0)),
            pl.BlockSpec((MAP_DIM,), lambda i: (0,)),
            pl.BlockSpec((MAP_DIM,), lambda i: (0,)),
        ],
        out_specs=pl.BlockSpec((BLK, MAP_DIM), lambda i: (i, 0)),
        out_shape=jax.ShapeDtypeStruct((N_MAP, MAP_DIM), jnp.float32),
    )(feats, acc, W_map, W_agt_pad, gn_gamma, gn_beta)


def kernel(feats, agent_feat, a2m_u, a2m_v, W_map, W_agt, gn_gamma, gn_beta):
    u = a2m_u.astype(jnp.int32)
    v = a2m_v.astype(jnp.int32)
    pad = E_PAD - N_EDGE
    u2 = jnp.concatenate([u, jnp.zeros((pad,), jnp.int32)])
    v2 = jnp.concatenate([v, jnp.full((pad,), N_MAP, jnp.int32)])
    uv = jnp.stack([u2.reshape(N_CHUNKS, CHUNK),
                    v2.reshape(N_CHUNKS, CHUNK)], axis=1)
    uv = uv.reshape(2 * N_CHUNKS, CHUNK)

    agent_tab = jnp.pad(agent_feat, ((0, 0), (0, AG_PAD - AG_DIM)))
    accs = _sc_accumulate(agent_tab, uv)
    acc = jnp.concatenate([accs[:HALF], accs[ACC_ROWS:ACC_ROWS + HALF]])
    W_agt_pad = jnp.pad(W_agt, ((0, 0), (0, AG_PAD - AG_DIM)))
    return _tc_finish(feats, acc, W_map, W_agt_pad, gn_gamma, gn_beta)
